# baseline (speedup 1.0000x reference)
import functools

import jax
import jax.numpy as jnp
from jax import lax
from jax.experimental import pallas as pl
from jax.experimental.pallas import tpu as pltpu
from jax.experimental.pallas import tpu_sc as plsc

_NUM_CORES = 2
_NUM_SUBCORES = 16
_NW = _NUM_CORES * _NUM_SUBCORES
_LANES = 16

_SRC = 45 * 90
_TGT = 90 * 180
_TGT_PAD = 16208
_ROW_CP = 4056
_SENT = 4056
_TBL_PAD = 4080
_LEAD = 32 * 69
_ROWS_PER_W = _LEAD // _NW
_NGRP = _TGT_PAD // _LANES


@jax.jit
def _interp(values1d, idx_pad, mask_pad):
    mesh = plsc.VectorSubcoreMesh(
        core_axis_name="c",
        subcore_axis_name="s",
        num_cores=_NUM_CORES,
        num_subcores=_NUM_SUBCORES,
    )

    @functools.partial(
        pl.kernel,
        out_type=jax.ShapeDtypeStruct((_LEAD * _TGT,), jnp.float32),
        mesh=mesh,
        compiler_params=pltpu.CompilerParams(needs_layout_passes=False),
        scratch_types=[
            pltpu.VMEM((_TGT_PAD,), jnp.int32),
            pltpu.VMEM((_TGT_PAD,), jnp.int32),
            pltpu.VMEM((_TBL_PAD,), jnp.float32),
            pltpu.VMEM((_TGT_PAD,), jnp.float32),
        ],
    )
    def body(values_hbm, idx_hbm, mask_hbm, out_hbm, idx_v, mask_v, tbl_v, row_v):
        wid = lax.axis_index("s") * _NUM_CORES + lax.axis_index("c")

        pltpu.sync_copy(idx_hbm, idx_v)
        pltpu.sync_copy(mask_hbm, mask_v)

        def fuse(g, carry):
            sl = pl.ds(g * _LANES, _LANES)
            iv = idx_v[sl]
            mv = mask_v[sl]
            idx_v[sl] = jnp.where(mv != 0, iv, jnp.full((_LANES,), _SENT, jnp.int32))
            return carry

        lax.fori_loop(0, _NGRP, fuse, 0)

        nan_vec = jnp.full((_LANES,), jnp.nan, jnp.float32)
        tbl_v[pl.ds(_TBL_PAD - 2 * _LANES, _LANES)] = nan_vec
        tbl_v[pl.ds(_TBL_PAD - _LANES, _LANES)] = nan_vec

        def per_row(r, carry):
            row = wid * _ROWS_PER_W + r
            src_start = row * _SRC
            aligned = pl.multiple_of((src_start // 8) * 8, 8)
            delta = src_start - aligned
            pltpu.sync_copy(
                values_hbm.at[pl.ds(aligned, _ROW_CP)],
                tbl_v.at[pl.ds(0, _ROW_CP)],
            )

            def gather(g, c2):
                sl = pl.ds(g * _LANES, _LANES)
                row_v[sl] = plsc.load_gather(tbl_v, [idx_v[sl] + delta])
                return c2

            lax.fori_loop(0, _NGRP, gather, 0)
            pltpu.sync_copy(
                row_v.at[pl.ds(0, _TGT)],
                out_hbm.at[pl.ds(pl.multiple_of(row * _TGT, 8), _TGT)],
            )
            return carry

        lax.fori_loop(0, _ROWS_PER_W, per_row, 0)

    return body(values1d, idx_pad, mask_pad)


def kernel(values, source_flat_index, valid_mask):
    lead = values.shape[:-2]
    v1 = values.reshape((-1,))
    idx = source_flat_index.astype(jnp.int32)
    mask = valid_mask.astype(jnp.int32)
    idx_pad = jnp.pad(idx, (0, _TGT_PAD - _TGT))
    mask_pad = jnp.pad(mask, (0, _TGT_PAD - _TGT))
    out = _interp(v1, idx_pad, mask_pad)
    return out.reshape(lead + (90, 180))


# speedup vs baseline: 1.4879x; 1.4879x over previous
import functools

import jax
import jax.numpy as jnp
from jax import lax
from jax.experimental import pallas as pl
from jax.experimental.pallas import tpu as pltpu
from jax.experimental.pallas import tpu_sc as plsc

_NUM_CORES = 2
_NUM_SUBCORES = 16
_NW = _NUM_CORES * _NUM_SUBCORES
_LANES = 16

_SRC = 45 * 90
_TGT = 90 * 180
_TGT_PAD = 16208
_ROW_CP = 4056
_SENT = 4056
_TBL_PAD = 4080
_LEAD = 32 * 69
_ROWS_PER_W = _LEAD // _NW
_NGRP = _TGT_PAD // _LANES


@jax.jit
def _interp(values1d, idx_pad, mask_pad):
    mesh = plsc.VectorSubcoreMesh(
        core_axis_name="c",
        subcore_axis_name="s",
        num_cores=_NUM_CORES,
        num_subcores=_NUM_SUBCORES,
    )

    @functools.partial(
        pl.kernel,
        out_type=jax.ShapeDtypeStruct((_LEAD * _TGT,), jnp.float32),
        mesh=mesh,
        compiler_params=pltpu.CompilerParams(needs_layout_passes=False),
        scratch_types=[
            pltpu.VMEM((_TGT_PAD,), jnp.int32),
            pltpu.VMEM((_TGT_PAD,), jnp.int32),
            pltpu.VMEM((_TBL_PAD,), jnp.float32),
            pltpu.VMEM((_TBL_PAD,), jnp.float32),
            pltpu.VMEM((_TGT_PAD,), jnp.float32),
            pltpu.VMEM((_TGT_PAD,), jnp.float32),
            pltpu.SemaphoreType.DMA,
            pltpu.SemaphoreType.DMA,
            pltpu.SemaphoreType.DMA,
            pltpu.SemaphoreType.DMA,
        ],
    )
    def body(values_hbm, idx_hbm, mask_hbm, out_hbm,
             idx_v, mask_v, tbl0_v, tbl1_v, out0_v, out1_v,
             sin0, sin1, sout0, sout1):
        wid = lax.axis_index("s") * _NUM_CORES + lax.axis_index("c")
        row0 = wid * _ROWS_PER_W
        tbl = (tbl0_v, tbl1_v)
        outb = (out0_v, out1_v)
        sin = (sin0, sin1)
        sout = (sout0, sout1)

        def row_src(row):
            src_start = row * _SRC
            aligned = pl.multiple_of((src_start // 8) * 8, 8)
            delta = src_start - aligned
            return values_hbm.at[pl.ds(aligned, _ROW_CP)], delta

        def fire_tbl(row, b):
            src, _ = row_src(row)
            pltpu.async_copy(src, tbl[b].at[pl.ds(0, _ROW_CP)], sin[b])

        def wait_tbl(row, b):
            src, _ = row_src(row)
            pltpu.make_async_copy(src, tbl[b].at[pl.ds(0, _ROW_CP)], sin[b]).wait()

        def out_dst(row):
            return out_hbm.at[pl.ds(pl.multiple_of(row * _TGT, 8), _TGT)]

        def fire_out(row, b):
            pltpu.async_copy(outb[b].at[pl.ds(0, _TGT)], out_dst(row), sout[b])

        def wait_out(row, b):
            pltpu.make_async_copy(
                outb[b].at[pl.ds(0, _TGT)], out_dst(row), sout[b]).wait()

        def gather_row(row, b):
            _, delta = row_src(row)

            @plsc.parallel_loop(0, _NGRP, unroll=8)
            def _(g):
                sl = pl.ds(g * _LANES, _LANES)
                outb[b][sl] = plsc.load_gather(tbl[b], [idx_v[sl] + delta])

        fire_tbl(row0, 0)
        pltpu.sync_copy(idx_hbm, idx_v)
        pltpu.sync_copy(mask_hbm, mask_v)

        @plsc.parallel_loop(0, _NGRP, unroll=4)
        def _(g):
            sl = pl.ds(g * _LANES, _LANES)
            iv = idx_v[sl]
            mv = mask_v[sl]
            idx_v[sl] = jnp.where(mv != 0, iv, jnp.full((_LANES,), _SENT, jnp.int32))

        nan_vec = jnp.full((_LANES,), jnp.nan, jnp.float32)
        for t in tbl:
            t[pl.ds(_TBL_PAD - 2 * _LANES, _LANES)] = nan_vec
            t[pl.ds(_TBL_PAD - _LANES, _LANES)] = nan_vec

        for b in (0, 1):
            row = row0 + b
            wait_tbl(row, b)
            fire_tbl(row + 1, 1 - b)
            gather_row(row, b)
            fire_out(row, b)

        @pl.loop(1, (_ROWS_PER_W - 1) // 2)
        def _(p):
            for b in (0, 1):
                row = row0 + 2 * p + b
                wait_tbl(row, b)
                fire_tbl(row + 1, 1 - b)
                wait_out(row - 2, b)
                gather_row(row, b)
                fire_out(row, b)

        last = row0 + _ROWS_PER_W - 1
        wait_tbl(last, 0)
        wait_out(last - 2, 0)
        gather_row(last, 0)
        fire_out(last, 0)
        wait_out(last - 1, 1)
        wait_out(last, 0)

    return body(values1d, idx_pad, mask_pad)


def kernel(values, source_flat_index, valid_mask):
    lead = values.shape[:-2]
    v1 = values.reshape((-1,))
    idx = source_flat_index.astype(jnp.int32)
    mask = valid_mask.astype(jnp.int32)
    idx_pad = jnp.pad(idx, (0, _TGT_PAD - _TGT))
    mask_pad = jnp.pad(mask, (0, _TGT_PAD - _TGT))
    out = _interp(v1, idx_pad, mask_pad)
    return out.reshape(lead + (90, 180))


# speedup vs baseline: 6.4049x; 4.3045x over previous
import functools

import jax
import jax.numpy as jnp
from jax import lax
from jax.experimental import pallas as pl
from jax.experimental.pallas import tpu as pltpu
from jax.experimental.pallas import tpu_sc as plsc

_NUM_CORES = 2
_NUM_SUBCORES = 16
_NW = _NUM_CORES * _NUM_SUBCORES
_L = 16

_B0, _B1 = 32, 69
_SH, _SW = 45, 90
_TH, _TW = 90, 180
_TGT = _TH * _TW
_TGT_PAD = _TGT + 8
_OCT = 8
_NUNIT = _B1 * (_B0 // _OCT)
_UPW = -(-_NUNIT // _NW)
_TCH = 10
_NCHUNK = _TH // _TCH
_SENT = _SH * 128
_MHALF = 8112


@jax.jit
def _interp(vt, idx_pad, mask_pad):
    mesh = plsc.VectorSubcoreMesh(
        core_axis_name="c",
        subcore_axis_name="s",
        num_cores=_NUM_CORES,
        num_subcores=_NUM_SUBCORES,
    )

    @functools.partial(
        pl.kernel,
        out_type=jax.ShapeDtypeStruct((_B1, _TH, _B0, _TW), jnp.float32),
        mesh=mesh,
        compiler_params=pltpu.CompilerParams(needs_layout_passes=False),
        scratch_types=[
            pltpu.VMEM((_TGT_PAD,), jnp.int32),
            pltpu.VMEM((_MHALF,), jnp.int32),
            pltpu.VMEM((_SH + 1, _OCT, _SW), jnp.float32),
            pltpu.VMEM((_TCH, _OCT, _TW), jnp.float32),
            pltpu.VMEM((_TCH, _OCT, _TW), jnp.float32),
            pltpu.SemaphoreType.DMA,
            pltpu.SemaphoreType.DMA,
        ],
    )
    def body(vt_hbm, idx_hbm, mask_hbm, out_hbm,
             idx_v, mask_v, tbl_v, st0_v, st1_v, sout0, sout1):
        wid = lax.axis_index("s") * _NUM_CORES + lax.axis_index("c")
        stage = (st0_v, st1_v)
        sout = (sout0, sout1)

        pltpu.sync_copy(idx_hbm, idx_v)
        sent_vec = jnp.full((_L,), _SENT, jnp.int32)

        for half, (hbase, hlen) in enumerate(((0, _MHALF), (_MHALF, _TGT_PAD - _MHALF))):
            pltpu.sync_copy(mask_hbm.at[pl.ds(hbase, hlen)],
                            mask_v.at[pl.ds(0, hlen)])

            @plsc.parallel_loop(0, hlen // _L, unroll=4)
            def _(g):
                sl = pl.ds(hbase + g * _L, _L)
                s = idx_v[sl]
                m = mask_v[pl.ds(g * _L, _L)]
                sh = (s * 46604) >> 22
                hw = (sh << 7) + (s - sh * 90)
                idx_v[sl] = jnp.where(m != 0, hw, sent_vec)

        nan_vec = jnp.full((_L,), jnp.nan, jnp.float32)
        for b in range(_OCT):
            tbl_v[_SH, b, pl.ds(0, _L)] = nan_vec

        def out_dst(b1, q, c):
            return out_hbm.at[b1, pl.ds(c * _TCH, _TCH), pl.ds(q * _OCT, _OCT), :]

        def fire(b1, q, c, buf):
            pltpu.async_copy(stage[buf], out_dst(b1, q, c), sout[buf])

        def absorb(buf):
            pltpu.make_async_copy(stage[buf], out_dst(0, 0, 0), sout[buf]).wait()

        def gather_chunk(c, buf):
            sbuf = stage[buf]

            @plsc.parallel_loop(0, _TCH * _OCT, unroll=2)
            def _(l):
                thl = l >> 3
                b0l = l & 7
                b0v = jnp.zeros((_L,), jnp.int32) + b0l
                rowbase = (c * _TCH + thl) * _TW
                for wi in range(12):
                    w0 = wi * _L if wi < 11 else _TW - _L
                    hw = idx_v[pl.ds(rowbase + w0, _L)]
                    h = hw >> 7
                    w = hw & 127
                    sbuf[thl, b0l, pl.ds(w0, _L)] = plsc.load_gather(
                        tbl_v, [h, b0v, w])

        def do_unit(i, u):
            b1 = u // 4
            q = u % 4
            pltpu.sync_copy(vt_hbm.at[b1, :, pl.ds(q * _OCT, _OCT), :],
                            tbl_v.at[pl.ds(0, _SH)])

            @pl.when(i > 0)
            def _():
                absorb(0)
            gather_chunk(0, 0)
            fire(b1, q, 0, 0)

            @pl.loop(0, (_NCHUNK - 1) // 2)
            def _(p):
                c1 = 2 * p + 1

                @pl.when((i > 0) | (p > 0))
                def _():
                    absorb(1)
                gather_chunk(c1, 1)
                fire(b1, q, c1, 1)

                absorb(0)
                gather_chunk(c1 + 1, 0)
                fire(b1, q, c1 + 1, 0)

        @pl.loop(0, _UPW)
        def _(i):
            u = wid + i * _NW

            @pl.when(u < _NUNIT)
            def _():
                do_unit(i, u)

        absorb(0)
        absorb(1)

    return body(vt, idx_pad, mask_pad)


def kernel(values, source_flat_index, valid_mask):
    vt = jnp.transpose(values, (1, 2, 0, 3))
    idx = jnp.pad(source_flat_index.astype(jnp.int32), (0, _TGT_PAD - _TGT))
    mask = jnp.pad(valid_mask.astype(jnp.int32), (0, _TGT_PAD - _TGT))
    out_t = _interp(vt, idx, mask)
    return jnp.transpose(out_t, (2, 0, 1, 3))
